# baseline (speedup 1.0000x reference)
import jax
import jax.numpy as jnp
from jax.experimental import pallas as pl
from jax.experimental.pallas import tpu as pltpu

_N_CHUNKS = 8


def _dma_copy(x_ref, o_ref, sems):
    rows = x_ref.shape[0]
    chunk = rows // _N_CHUNKS
    copies = [
        pltpu.make_async_copy(
            x_ref.at[pl.ds(i * chunk, chunk)],
            o_ref.at[pl.ds(i * chunk, chunk)],
            sems.at[i],
        )
        for i in range(_N_CHUNKS)
    ]
    for c in copies:
        c.start()
    for c in copies:
        c.wait()


def kernel(x):
    b, c, h, w = x.shape
    flat = x.reshape(b * c * h // 2, w * 2)
    out = pl.pallas_call(
        _dma_copy,
        in_specs=[pl.BlockSpec(memory_space=pltpu.MemorySpace.HBM)],
        out_specs=pl.BlockSpec(memory_space=pltpu.MemorySpace.HBM),
        out_shape=jax.ShapeDtypeStruct(flat.shape, flat.dtype),
        scratch_shapes=[pltpu.SemaphoreType.DMA((_N_CHUNKS,))],
    )(flat)
    return out.reshape(x.shape)


# speedup vs baseline: 11.4604x; 11.4604x over previous
import jax
import jax.numpy as jnp
from jax.experimental import pallas as pl
from jax.experimental.pallas import tpu as pltpu

_BLOCK_ROWS = 512


def _copy_block(x_ref, o_ref):
    o_ref[...] = x_ref[...]


def kernel(x):
    b, c, h, w = x.shape
    flat = x.reshape(b * c * h // 2, w * 2)
    rows, cols = flat.shape
    grid = (rows // _BLOCK_ROWS,)
    out = pl.pallas_call(
        _copy_block,
        grid=grid,
        in_specs=[pl.BlockSpec((_BLOCK_ROWS, cols), lambda i: (i, 0))],
        out_specs=pl.BlockSpec((_BLOCK_ROWS, cols), lambda i: (i, 0)),
        out_shape=jax.ShapeDtypeStruct(flat.shape, flat.dtype),
    )(flat)
    return out.reshape(x.shape)


# speedup vs baseline: 11.9036x; 1.0387x over previous
import jax
import jax.numpy as jnp
from jax.experimental import pallas as pl
from jax.experimental.pallas import tpu as pltpu

_SLOTS = 16
_CHUNKS = 24


def _dma_copy(x_ref, o_ref, scratch, in_sems, out_sems):
    rows = x_ref.shape[0]
    ch = rows // _CHUNKS

    def in_copy(i):
        return pltpu.make_async_copy(
            x_ref.at[pl.ds(i * ch, ch)], scratch.at[i % _SLOTS],
            in_sems.at[i % _SLOTS])

    def out_copy(i):
        return pltpu.make_async_copy(
            scratch.at[i % _SLOTS], o_ref.at[pl.ds(i * ch, ch)],
            out_sems.at[i % _SLOTS])

    for i in range(_SLOTS):
        in_copy(i).start()
    for i in range(_CHUNKS):
        in_copy(i).wait()
        out_copy(i).start()
        j = i + _SLOTS
        if j < _CHUNKS:
            out_copy(i).wait()
            in_copy(j).start()
    for i in range(max(_CHUNKS - _SLOTS, 0), _CHUNKS):
        out_copy(i).wait()


def kernel(x):
    b, c, h, w = x.shape
    flat = x.reshape(b * c * h // 2, w * 2)
    rows, cols = flat.shape
    ch = rows // _CHUNKS
    out = pl.pallas_call(
        _dma_copy,
        in_specs=[pl.BlockSpec(memory_space=pltpu.MemorySpace.HBM)],
        out_specs=pl.BlockSpec(memory_space=pltpu.MemorySpace.HBM),
        out_shape=jax.ShapeDtypeStruct(flat.shape, flat.dtype),
        scratch_shapes=[
            pltpu.VMEM((_SLOTS, ch, cols), jnp.float32),
            pltpu.SemaphoreType.DMA((_SLOTS,)),
            pltpu.SemaphoreType.DMA((_SLOTS,)),
        ],
    )(flat)
    return out.reshape(x.shape)
